# baseline (speedup 1.0000x reference)
import functools

import jax
import jax.numpy as jnp
from jax.experimental import pallas as pl
from jax.experimental.pallas import tpu as pltpu


def _knn_mlp1_kernel(rel_ref, x1_ref, x2t_ref, p1_ref, p2_ref,
                     w1t_ref, b1_ref, z1_ref, s1_ref, q1_ref, *, tn, s, d, nb):
    bi = pl.program_id(0)
    ni = pl.program_id(1)

    x1 = x1_ref[0]
    x2t = x2t_ref[0]
    sq1 = jnp.sum(x1 * x1, axis=1, keepdims=True)
    sq2 = jnp.sum(x2t * x2t, axis=0, keepdims=True)
    dist = -2.0 * jnp.dot(x1, x2t, preferred_element_type=jnp.float32) + sq1 + sq2

    rw0 = rel_ref[0]
    rb = rel_ref[4]
    proj1 = (x1[:, 0:1] * rel_ref[1] + x1[:, 1:2] * rel_ref[2]
             + x1[:, 2:3] * rel_ref[3])
    proj2 = (x2t[0:1, :] * rel_ref[1] + x2t[1:2, :] * rel_ref[2]
             + x2t[2:3, :] * rel_ref[3])

    iota = jax.lax.broadcasted_iota(jnp.int32, (tn, s), 1).astype(jnp.float32)
    cur = dist
    m_acc = jnp.zeros((tn, s), jnp.float32)
    for k in range(3):
        mval = jnp.min(cur, axis=1, keepdims=True)
        cand = jnp.where(cur == mval, iota, jnp.float32(s))
        amin = jnp.min(cand, axis=1, keepdims=True)
        sel = cand == amin
        pg = jnp.sum(jnp.where(sel, proj2, 0.0), axis=1, keepdims=True)
        wk = jax.nn.sigmoid(rw0 * mval + proj1 - pg + rb) * (1.0 / 3.0)
        m_acc = jnp.where(sel, wk, m_acc)
        if k < 2:
            cur = jnp.where(sel, jnp.float32(jnp.inf), cur)

    interp = jnp.dot(m_acc, p2_ref[0], preferred_element_type=jnp.float32)
    z1 = (jnp.dot(p1_ref[0], w1t_ref[:d], preferred_element_type=jnp.float32)
          + jnp.dot(interp.astype(jnp.bfloat16), w1t_ref[d:],
                    preferred_element_type=jnp.float32)
          + b1_ref[...])
    z1_ref[...] = z1.astype(jnp.bfloat16)

    @pl.when(jnp.logical_and(bi == 0, ni == 0))
    def _():
        s1_ref[...] = jnp.zeros_like(s1_ref)
        q1_ref[...] = jnp.zeros_like(q1_ref)

    s1_ref[...] += jnp.sum(z1, axis=0, keepdims=True)
    q1_ref[...] += jnp.sum(z1 * z1, axis=0, keepdims=True)


def _mlp2_kernel(z1_ref, s1_ref, q1_ref, g1_ref, bt1_ref, w2t_ref, b2_ref,
                 z2_ref, s2_ref, q2_ref, *, m):
    mu = s1_ref[...] * (1.0 / m)
    var = q1_ref[...] * (1.0 / m) - mu * mu
    scale = g1_ref[...] * jax.lax.rsqrt(var + 1e-5)
    a = jnp.maximum((z1_ref[...].astype(jnp.float32) - mu) * scale + bt1_ref[...],
                    0.0)
    z2 = jnp.dot(a, w2t_ref[...], preferred_element_type=jnp.float32) + b2_ref[...]
    z2_ref[...] = z2.astype(jnp.bfloat16)

    @pl.when(pl.program_id(0) == 0)
    def _():
        s2_ref[...] = jnp.zeros_like(s2_ref)
        q2_ref[...] = jnp.zeros_like(q2_ref)

    s2_ref[...] += jnp.sum(z2, axis=0, keepdims=True)
    q2_ref[...] += jnp.sum(z2 * z2, axis=0, keepdims=True)


def _bn2_kernel(z2_ref, s2_ref, q2_ref, g2_ref, bt2_ref, out_ref, *, m):
    mu = s2_ref[...] * (1.0 / m)
    var = q2_ref[...] * (1.0 / m) - mu * mu
    scale = g2_ref[...] * jax.lax.rsqrt(var + 1e-5)
    out_ref[...] = jnp.maximum(
        (z2_ref[...].astype(jnp.float32) - mu) * scale + bt2_ref[...], 0.0)


def kernel(xyz1, xyz2, points1, points2, rel_w, rel_b, w1, b1, g1, beta1,
           w2, b2, g2, beta2):
    B, N, _ = xyz1.shape
    S = xyz2.shape[1]
    D = points2.shape[2]
    TN = min(2048, N)
    NB = N // TN
    M = B * N

    x2t = jnp.transpose(xyz2, (0, 2, 1))
    rel = jnp.concatenate([rel_w, rel_b])
    w1t = w1.T.astype(jnp.bfloat16)
    w2t = w2.T
    p1h = points1.astype(jnp.bfloat16)
    b1r = b1.reshape(1, D)
    g1r = g1.reshape(1, D)
    bt1 = beta1.reshape(1, D)
    b2r = b2.reshape(1, D)
    g2r = g2.reshape(1, D)
    bt2 = beta2.reshape(1, D)

    z1, s1, q1 = pl.pallas_call(
        functools.partial(_knn_mlp1_kernel, tn=TN, s=S, d=D, nb=NB),
        grid=(B, NB),
        in_specs=[
            pl.BlockSpec(memory_space=pltpu.SMEM),
            pl.BlockSpec((1, TN, 3), lambda b, n: (b, n, 0)),
            pl.BlockSpec((1, 3, S), lambda b, n: (b, 0, 0)),
            pl.BlockSpec((1, TN, D), lambda b, n: (b, n, 0)),
            pl.BlockSpec((1, S, D), lambda b, n: (b, 0, 0)),
            pl.BlockSpec((2 * D, D), lambda b, n: (0, 0)),
            pl.BlockSpec((1, D), lambda b, n: (0, 0)),
        ],
        out_specs=[
            pl.BlockSpec((TN, D), lambda b, n: (b * NB + n, 0)),
            pl.BlockSpec((1, D), lambda b, n: (0, 0)),
            pl.BlockSpec((1, D), lambda b, n: (0, 0)),
        ],
        out_shape=[
            jax.ShapeDtypeStruct((M, D), jnp.bfloat16),
            jax.ShapeDtypeStruct((1, D), jnp.float32),
            jax.ShapeDtypeStruct((1, D), jnp.float32),
        ],
    )(rel, xyz1, x2t, p1h, points2, w1t, b1r)

    TN2 = min(4096, M)
    z2, s2, q2 = pl.pallas_call(
        functools.partial(_mlp2_kernel, m=float(M)),
        grid=(M // TN2,),
        in_specs=[
            pl.BlockSpec((TN2, D), lambda i: (i, 0)),
            pl.BlockSpec((1, D), lambda i: (0, 0)),
            pl.BlockSpec((1, D), lambda i: (0, 0)),
            pl.BlockSpec((1, D), lambda i: (0, 0)),
            pl.BlockSpec((1, D), lambda i: (0, 0)),
            pl.BlockSpec((D, D), lambda i: (0, 0)),
            pl.BlockSpec((1, D), lambda i: (0, 0)),
        ],
        out_specs=[
            pl.BlockSpec((TN2, D), lambda i: (i, 0)),
            pl.BlockSpec((1, D), lambda i: (0, 0)),
            pl.BlockSpec((1, D), lambda i: (0, 0)),
        ],
        out_shape=[
            jax.ShapeDtypeStruct((M, D), jnp.bfloat16),
            jax.ShapeDtypeStruct((1, D), jnp.float32),
            jax.ShapeDtypeStruct((1, D), jnp.float32),
        ],
    )(z1, s1, q1, g1r, bt1, w2t, b2r)

    TN3 = min(4096, M)
    out = pl.pallas_call(
        functools.partial(_bn2_kernel, m=float(M)),
        grid=(M // TN3,),
        in_specs=[
            pl.BlockSpec((TN3, D), lambda i: (i, 0)),
            pl.BlockSpec((1, D), lambda i: (0, 0)),
            pl.BlockSpec((1, D), lambda i: (0, 0)),
            pl.BlockSpec((1, D), lambda i: (0, 0)),
            pl.BlockSpec((1, D), lambda i: (0, 0)),
        ],
        out_specs=pl.BlockSpec((TN3, D), lambda i: (i, 0)),
        out_shape=jax.ShapeDtypeStruct((M, D), jnp.float32),
    )(z2, s2, q2, g2r, bt2)

    return out.reshape(B, N, D)


# speedup vs baseline: 1.0598x; 1.0598x over previous
import functools

import jax
import jax.numpy as jnp
from jax.experimental import pallas as pl
from jax.experimental.pallas import tpu as pltpu


def _knn_mlp1_kernel(rel_ref, x1_ref, x2t_ref, p1_ref, p2_ref,
                     w1t_ref, b1_ref, z1_ref, s1_ref, q1_ref, *, tn, s, d, nb):
    bi = pl.program_id(0)
    ni = pl.program_id(1)

    x1 = x1_ref[0]
    x2t = x2t_ref[0]
    sq1 = jnp.sum(x1 * x1, axis=1, keepdims=True)
    sq2 = jnp.sum(x2t * x2t, axis=0, keepdims=True)
    dist = -2.0 * jnp.dot(x1, x2t, preferred_element_type=jnp.float32) + sq1 + sq2

    rw0 = rel_ref[0]
    rb = rel_ref[4]
    proj1 = (x1[:, 0:1] * rel_ref[1] + x1[:, 1:2] * rel_ref[2]
             + x1[:, 2:3] * rel_ref[3])
    proj2 = (x2t[0:1, :] * rel_ref[1] + x2t[1:2, :] * rel_ref[2]
             + x2t[2:3, :] * rel_ref[3])

    iota = jax.lax.broadcasted_iota(jnp.int32, (tn, s), 1).astype(jnp.float32)
    cur = dist
    m_acc = jnp.zeros((tn, s), jnp.float32)
    for k in range(3):
        mval = jnp.min(cur, axis=1, keepdims=True)
        cand = jnp.where(cur == mval, iota, jnp.float32(s))
        amin = jnp.min(cand, axis=1, keepdims=True)
        sel = cand == amin
        pg = jnp.sum(jnp.where(sel, proj2, 0.0), axis=1, keepdims=True)
        wk = jax.nn.sigmoid(rw0 * mval + proj1 - pg + rb) * (1.0 / 3.0)
        m_acc = jnp.where(sel, wk, m_acc)
        if k < 2:
            cur = jnp.where(sel, jnp.float32(jnp.inf), cur)

    interp = jnp.dot(m_acc, p2_ref[0], preferred_element_type=jnp.float32)
    z1 = (jnp.dot(p1_ref[0], w1t_ref[:d], preferred_element_type=jnp.float32)
          + jnp.dot(interp, w1t_ref[d:], preferred_element_type=jnp.float32)
          + b1_ref[...])
    z1_ref[...] = z1.astype(jnp.bfloat16)

    @pl.when(jnp.logical_and(bi == 0, ni == 0))
    def _():
        s1_ref[...] = jnp.zeros_like(s1_ref)
        q1_ref[...] = jnp.zeros_like(q1_ref)

    s1_ref[...] += jnp.sum(z1, axis=0, keepdims=True)
    q1_ref[...] += jnp.sum(z1 * z1, axis=0, keepdims=True)


def _mlp2_kernel(z1_ref, s1_ref, q1_ref, g1_ref, bt1_ref, w2t_ref, b2_ref,
                 z2_ref, s2_ref, q2_ref, *, m):
    mu = s1_ref[...] * (1.0 / m)
    var = q1_ref[...] * (1.0 / m) - mu * mu
    scale = g1_ref[...] * jax.lax.rsqrt(var + 1e-5)
    a = jnp.maximum((z1_ref[...].astype(jnp.float32) - mu) * scale + bt1_ref[...],
                    0.0)
    z2 = jnp.dot(a, w2t_ref[...], preferred_element_type=jnp.float32) + b2_ref[...]
    z2_ref[...] = z2.astype(jnp.bfloat16)

    @pl.when(pl.program_id(0) == 0)
    def _():
        s2_ref[...] = jnp.zeros_like(s2_ref)
        q2_ref[...] = jnp.zeros_like(q2_ref)

    s2_ref[...] += jnp.sum(z2, axis=0, keepdims=True)
    q2_ref[...] += jnp.sum(z2 * z2, axis=0, keepdims=True)


def _bn2_kernel(z2_ref, s2_ref, q2_ref, g2_ref, bt2_ref, out_ref, *, m):
    mu = s2_ref[...] * (1.0 / m)
    var = q2_ref[...] * (1.0 / m) - mu * mu
    scale = g2_ref[...] * jax.lax.rsqrt(var + 1e-5)
    out_ref[...] = jnp.maximum(
        (z2_ref[...].astype(jnp.float32) - mu) * scale + bt2_ref[...], 0.0)


def kernel(xyz1, xyz2, points1, points2, rel_w, rel_b, w1, b1, g1, beta1,
           w2, b2, g2, beta2):
    B, N, _ = xyz1.shape
    S = xyz2.shape[1]
    D = points2.shape[2]
    TN = min(2048, N)
    NB = N // TN
    M = B * N

    x2t = jnp.transpose(xyz2, (0, 2, 1))
    rel = jnp.concatenate([rel_w, rel_b])
    w1t = w1.T
    w2t = w2.T
    b1r = b1.reshape(1, D)
    g1r = g1.reshape(1, D)
    bt1 = beta1.reshape(1, D)
    b2r = b2.reshape(1, D)
    g2r = g2.reshape(1, D)
    bt2 = beta2.reshape(1, D)

    z1, s1, q1 = pl.pallas_call(
        functools.partial(_knn_mlp1_kernel, tn=TN, s=S, d=D, nb=NB),
        grid=(B, NB),
        in_specs=[
            pl.BlockSpec(memory_space=pltpu.SMEM),
            pl.BlockSpec((1, TN, 3), lambda b, n: (b, n, 0)),
            pl.BlockSpec((1, 3, S), lambda b, n: (b, 0, 0)),
            pl.BlockSpec((1, TN, D), lambda b, n: (b, n, 0)),
            pl.BlockSpec((1, S, D), lambda b, n: (b, 0, 0)),
            pl.BlockSpec((2 * D, D), lambda b, n: (0, 0)),
            pl.BlockSpec((1, D), lambda b, n: (0, 0)),
        ],
        out_specs=[
            pl.BlockSpec((TN, D), lambda b, n: (b * NB + n, 0)),
            pl.BlockSpec((1, D), lambda b, n: (0, 0)),
            pl.BlockSpec((1, D), lambda b, n: (0, 0)),
        ],
        out_shape=[
            jax.ShapeDtypeStruct((M, D), jnp.bfloat16),
            jax.ShapeDtypeStruct((1, D), jnp.float32),
            jax.ShapeDtypeStruct((1, D), jnp.float32),
        ],
    )(rel, xyz1, x2t, points1, points2, w1t, b1r)

    TN2 = min(8192, M)
    z2, s2, q2 = pl.pallas_call(
        functools.partial(_mlp2_kernel, m=float(M)),
        grid=(M // TN2,),
        in_specs=[
            pl.BlockSpec((TN2, D), lambda i: (i, 0)),
            pl.BlockSpec((1, D), lambda i: (0, 0)),
            pl.BlockSpec((1, D), lambda i: (0, 0)),
            pl.BlockSpec((1, D), lambda i: (0, 0)),
            pl.BlockSpec((1, D), lambda i: (0, 0)),
            pl.BlockSpec((D, D), lambda i: (0, 0)),
            pl.BlockSpec((1, D), lambda i: (0, 0)),
        ],
        out_specs=[
            pl.BlockSpec((TN2, D), lambda i: (i, 0)),
            pl.BlockSpec((1, D), lambda i: (0, 0)),
            pl.BlockSpec((1, D), lambda i: (0, 0)),
        ],
        out_shape=[
            jax.ShapeDtypeStruct((M, D), jnp.bfloat16),
            jax.ShapeDtypeStruct((1, D), jnp.float32),
            jax.ShapeDtypeStruct((1, D), jnp.float32),
        ],
    )(z1, s1, q1, g1r, bt1, w2t, b2r)

    TN3 = min(8192, M)
    out = pl.pallas_call(
        functools.partial(_bn2_kernel, m=float(M)),
        grid=(M // TN3,),
        in_specs=[
            pl.BlockSpec((TN3, D), lambda i: (i, 0)),
            pl.BlockSpec((1, D), lambda i: (0, 0)),
            pl.BlockSpec((1, D), lambda i: (0, 0)),
            pl.BlockSpec((1, D), lambda i: (0, 0)),
            pl.BlockSpec((1, D), lambda i: (0, 0)),
        ],
        out_specs=pl.BlockSpec((TN3, D), lambda i: (i, 0)),
        out_shape=jax.ShapeDtypeStruct((M, D), jnp.float32),
    )(z2, s2, q2, g2r, bt2)

    return out.reshape(B, N, D)


# speedup vs baseline: 1.0620x; 1.0020x over previous
import functools

import jax
import jax.numpy as jnp
from jax.experimental import pallas as pl
from jax.experimental.pallas import tpu as pltpu


def _knn_mlp1_kernel(rel_ref, x1_ref, x2t_ref, p1_ref, p2_ref,
                     w1t_ref, b1_ref, z1_ref, s1_ref, q1_ref, *, tn, s, d, nb):
    bi = pl.program_id(0)
    ni = pl.program_id(1)

    x1 = x1_ref[0]
    x2t = x2t_ref[0]
    sq1 = jnp.sum(x1 * x1, axis=1, keepdims=True)
    sq2 = jnp.sum(x2t * x2t, axis=0, keepdims=True)
    dist = -2.0 * jnp.dot(x1, x2t, preferred_element_type=jnp.float32) + sq1 + sq2

    rw0 = rel_ref[0]
    rb = rel_ref[4]
    proj1 = (x1[:, 0:1] * rel_ref[1] + x1[:, 1:2] * rel_ref[2]
             + x1[:, 2:3] * rel_ref[3])
    proj2 = (x2t[0:1, :] * rel_ref[1] + x2t[1:2, :] * rel_ref[2]
             + x2t[2:3, :] * rel_ref[3])

    iota = jax.lax.broadcasted_iota(jnp.int32, (tn, s), 1).astype(jnp.float32)
    cur = dist
    m_acc = jnp.zeros((tn, s), jnp.float32)
    for k in range(3):
        mval = jnp.min(cur, axis=1, keepdims=True)
        cand = jnp.where(cur == mval, iota, jnp.float32(s))
        amin = jnp.min(cand, axis=1, keepdims=True)
        sel = cand == amin
        pg = jnp.sum(jnp.where(sel, proj2, 0.0), axis=1, keepdims=True)
        wk = jax.nn.sigmoid(rw0 * mval + proj1 - pg + rb) * (1.0 / 3.0)
        m_acc = jnp.where(sel, wk, m_acc)
        if k < 2:
            cur = jnp.where(sel, jnp.float32(jnp.inf), cur)

    interp = jnp.dot(m_acc, p2_ref[0], preferred_element_type=jnp.float32)
    z1 = (jnp.dot(p1_ref[0], w1t_ref[:d], preferred_element_type=jnp.float32)
          + jnp.dot(interp, w1t_ref[d:], preferred_element_type=jnp.float32)
          + b1_ref[...])
    z1_ref[...] = z1.astype(jnp.bfloat16)

    @pl.when(jnp.logical_and(bi == 0, ni == 0))
    def _():
        s1_ref[...] = jnp.zeros_like(s1_ref)
        q1_ref[...] = jnp.zeros_like(q1_ref)

    s1_ref[...] += jnp.sum(z1, axis=0, keepdims=True)
    q1_ref[...] += jnp.sum(z1 * z1, axis=0, keepdims=True)


def _mlp2_kernel(z1_ref, s1_ref, q1_ref, g1_ref, bt1_ref, w2t_ref, b2_ref,
                 z2_ref, s2_ref, q2_ref, *, m):
    mu = s1_ref[...] * (1.0 / m)
    var = q1_ref[...] * (1.0 / m) - mu * mu
    scale = g1_ref[...] * jax.lax.rsqrt(var + 1e-5)
    a = jnp.maximum((z1_ref[...].astype(jnp.float32) - mu) * scale + bt1_ref[...],
                    0.0)
    z2 = jnp.dot(a, w2t_ref[...], preferred_element_type=jnp.float32) + b2_ref[...]
    z2_ref[...] = z2.astype(jnp.bfloat16)

    @pl.when(pl.program_id(0) == 0)
    def _():
        s2_ref[...] = jnp.zeros_like(s2_ref)
        q2_ref[...] = jnp.zeros_like(q2_ref)

    s2_ref[...] += jnp.sum(z2, axis=0, keepdims=True)
    q2_ref[...] += jnp.sum(z2 * z2, axis=0, keepdims=True)


def _bn2_kernel(z2_ref, s2_ref, q2_ref, g2_ref, bt2_ref, out_ref, *, m):
    mu = s2_ref[...] * (1.0 / m)
    var = q2_ref[...] * (1.0 / m) - mu * mu
    scale = g2_ref[...] * jax.lax.rsqrt(var + 1e-5)
    out_ref[...] = jnp.maximum(
        (z2_ref[...].astype(jnp.float32) - mu) * scale + bt2_ref[...], 0.0)


def kernel(xyz1, xyz2, points1, points2, rel_w, rel_b, w1, b1, g1, beta1,
           w2, b2, g2, beta2):
    B, N, _ = xyz1.shape
    S = xyz2.shape[1]
    D = points2.shape[2]
    TN = min(2048, N)
    NB = N // TN
    M = B * N

    x2t = jnp.transpose(xyz2, (0, 2, 1))
    rel = jnp.concatenate([rel_w, rel_b])
    w1t = w1.T
    w2t = w2.T
    b1r = b1.reshape(1, D)
    g1r = g1.reshape(1, D)
    bt1 = beta1.reshape(1, D)
    b2r = b2.reshape(1, D)
    g2r = g2.reshape(1, D)
    bt2 = beta2.reshape(1, D)

    z1, s1, q1 = pl.pallas_call(
        functools.partial(_knn_mlp1_kernel, tn=TN, s=S, d=D, nb=NB),
        grid=(B, NB),
        compiler_params=pltpu.CompilerParams(
            vmem_limit_bytes=120 * 1024 * 1024),
        in_specs=[
            pl.BlockSpec(memory_space=pltpu.SMEM),
            pl.BlockSpec((1, TN, 3), lambda b, n: (b, n, 0)),
            pl.BlockSpec((1, 3, S), lambda b, n: (b, 0, 0)),
            pl.BlockSpec((1, TN, D), lambda b, n: (b, n, 0)),
            pl.BlockSpec((1, S, D), lambda b, n: (b, 0, 0)),
            pl.BlockSpec((2 * D, D), lambda b, n: (0, 0)),
            pl.BlockSpec((1, D), lambda b, n: (0, 0)),
        ],
        out_specs=[
            pl.BlockSpec((TN, D), lambda b, n: (b * NB + n, 0)),
            pl.BlockSpec((1, D), lambda b, n: (0, 0)),
            pl.BlockSpec((1, D), lambda b, n: (0, 0)),
        ],
        out_shape=[
            jax.ShapeDtypeStruct((M, D), jnp.bfloat16),
            jax.ShapeDtypeStruct((1, D), jnp.float32),
            jax.ShapeDtypeStruct((1, D), jnp.float32),
        ],
    )(rel, xyz1, x2t, points1, points2, w1t, b1r)

    TN2 = min(8192, M)
    z2, s2, q2 = pl.pallas_call(
        functools.partial(_mlp2_kernel, m=float(M)),
        grid=(M // TN2,),
        in_specs=[
            pl.BlockSpec((TN2, D), lambda i: (i, 0)),
            pl.BlockSpec((1, D), lambda i: (0, 0)),
            pl.BlockSpec((1, D), lambda i: (0, 0)),
            pl.BlockSpec((1, D), lambda i: (0, 0)),
            pl.BlockSpec((1, D), lambda i: (0, 0)),
            pl.BlockSpec((D, D), lambda i: (0, 0)),
            pl.BlockSpec((1, D), lambda i: (0, 0)),
        ],
        out_specs=[
            pl.BlockSpec((TN2, D), lambda i: (i, 0)),
            pl.BlockSpec((1, D), lambda i: (0, 0)),
            pl.BlockSpec((1, D), lambda i: (0, 0)),
        ],
        out_shape=[
            jax.ShapeDtypeStruct((M, D), jnp.bfloat16),
            jax.ShapeDtypeStruct((1, D), jnp.float32),
            jax.ShapeDtypeStruct((1, D), jnp.float32),
        ],
    )(z1, s1, q1, g1r, bt1, w2t, b2r)

    TN3 = min(8192, M)
    out = pl.pallas_call(
        functools.partial(_bn2_kernel, m=float(M)),
        grid=(M // TN3,),
        in_specs=[
            pl.BlockSpec((TN3, D), lambda i: (i, 0)),
            pl.BlockSpec((1, D), lambda i: (0, 0)),
            pl.BlockSpec((1, D), lambda i: (0, 0)),
            pl.BlockSpec((1, D), lambda i: (0, 0)),
            pl.BlockSpec((1, D), lambda i: (0, 0)),
        ],
        out_specs=pl.BlockSpec((TN3, D), lambda i: (i, 0)),
        out_shape=jax.ShapeDtypeStruct((M, D), jnp.float32),
    )(z2, s2, q2, g2r, bt2)

    return out.reshape(B, N, D)
